# baseline (speedup 1.0000x reference)
import functools

import jax
import jax.numpy as jnp
from jax.experimental import pallas as pl
from jax.experimental.pallas import tpu as pltpu


def _proj_body(x_ref, wb_ref, wr_ref, xw_ref):
    x = x_ref[...]
    wr = wr_ref[...]
    nb = wb_ref.shape[0]
    nr = wr.shape[0]
    xb = [
        jnp.dot(x, wb_ref[b], preferred_element_type=jnp.float32)
        for b in range(nb)
    ]
    for r in range(nr):
        acc = wr[r, 0] * xb[0]
        for b in range(1, nb):
            acc = acc + wr[r, b] * xb[b]
        xw_ref[r] = acc


def _project(x, w_bases, w_rel):
    nbasis, _, f_out = w_bases.shape
    nrel = w_rel.shape[0]
    n = x.shape[0]
    return pl.pallas_call(
        _proj_body,
        out_shape=jax.ShapeDtypeStruct((nrel, n, f_out), jnp.float32),
    )(x, w_bases, w_rel)


def _spmm_body(a_ref, xw_ref, b_ref, o_ref, *, nrel, relu):
    r = pl.program_id(1)
    acc = jnp.dot(a_ref[0], xw_ref[0], preferred_element_type=jnp.float32)

    @pl.when(r == 0)
    def _init():
        o_ref[...] = acc + b_ref[...]

    @pl.when(r > 0)
    def _accum():
        o_ref[...] += acc

    if relu:
        @pl.when(r == nrel - 1)
        def _relu():
            o_ref[...] = jnp.maximum(o_ref[...], 0.0)


def _spmm(A, xw, bias, relu, block_rows=512):
    nrel, n, _ = A.shape
    f_out = xw.shape[2]
    bias2d = bias.reshape(1, f_out)
    grid = (n // block_rows, nrel)
    return pl.pallas_call(
        functools.partial(_spmm_body, nrel=nrel, relu=relu),
        grid=grid,
        in_specs=[
            pl.BlockSpec((1, block_rows, n), lambda i, r: (r, i, 0)),
            pl.BlockSpec((1, n, f_out), lambda i, r: (r, 0, 0)),
            pl.BlockSpec((1, f_out), lambda i, r: (0, 0)),
        ],
        out_specs=pl.BlockSpec((block_rows, f_out), lambda i, r: (i, 0)),
        out_shape=jax.ShapeDtypeStruct((n, f_out), jnp.float32),
        compiler_params=pltpu.CompilerParams(
            dimension_semantics=("parallel", "arbitrary"),
        ),
    )(A, xw, bias2d)


def kernel(A, x, w_bases1, w_rel1, bias1, w_bases2, w_rel2, bias2):
    xw1 = _project(x, w_bases1, w_rel1)
    h = _spmm(A, xw1, bias1, relu=True)
    hw2 = _project(h, w_bases2, w_rel2)
    return _spmm(A, hw2, bias2, relu=False)


# speedup vs baseline: 1.2373x; 1.2373x over previous
import functools

import jax
import jax.numpy as jnp
from jax.experimental import pallas as pl
from jax.experimental.pallas import tpu as pltpu


def _proj_body(x_ref, wb_ref, wr_ref, xw_ref):
    x = x_ref[...]
    wr = wr_ref[...]
    nb = wb_ref.shape[0]
    nr = wr.shape[0]
    xb = [
        jnp.dot(x, wb_ref[b], preferred_element_type=jnp.float32)
        for b in range(nb)
    ]
    for r in range(nr):
        acc = wr[r, 0] * xb[0]
        for b in range(1, nb):
            acc = acc + wr[r, b] * xb[b]
        xw_ref[r] = acc


def _project(x, w_bases, w_rel):
    nbasis, _, f_out = w_bases.shape
    nrel = w_rel.shape[0]
    n = x.shape[0]
    return pl.pallas_call(
        _proj_body,
        out_shape=jax.ShapeDtypeStruct((nrel, n, f_out), jnp.float32),
    )(x, w_bases, w_rel)


def _proj_basis_body(x_ref, wb_ref, hb_ref):
    nb = wb_ref.shape[0]
    x = x_ref[...]
    for b in range(nb):
        hb_ref[b] = jnp.dot(
            x, wb_ref[b], preferred_element_type=jnp.float32
        ).astype(jnp.bfloat16)


def _project_basis(x, w_bases):
    nbasis, _, f_out = w_bases.shape
    n = x.shape[0]
    return pl.pallas_call(
        _proj_basis_body,
        out_shape=jax.ShapeDtypeStruct((nbasis, n, f_out), jnp.bfloat16),
    )(x, w_bases)


def _layer1_body(a_ref, xw_ref, wr2_ref, b_ref, h_ref, ab2_ref, *, nrel, nbasis):
    acc = jnp.dot(a_ref[0], xw_ref[0], preferred_element_type=jnp.float32)
    for r in range(1, nrel):
        acc += jnp.dot(a_ref[r], xw_ref[r], preferred_element_type=jnp.float32)
    h_ref[...] = jnp.maximum(acc + b_ref[...], 0.0)
    wr2 = wr2_ref[...]
    for b in range(nbasis):
        combo = wr2[0, b] * a_ref[0]
        for r in range(1, nrel):
            combo += wr2[r, b] * a_ref[r]
        ab2_ref[b] = combo.astype(jnp.bfloat16)


def _layer1(A, xw1, w_rel2, bias1, block_rows=256):
    nrel, n, _ = A.shape
    f_out = xw1.shape[2]
    nbasis = w_rel2.shape[1]
    bias2d = bias1.reshape(1, f_out)
    grid = (n // block_rows,)
    return pl.pallas_call(
        functools.partial(_layer1_body, nrel=nrel, nbasis=nbasis),
        grid=grid,
        in_specs=[
            pl.BlockSpec((nrel, block_rows, n), lambda i: (0, i, 0)),
            pl.BlockSpec((nrel, n, f_out), lambda i: (0, 0, 0)),
            pl.BlockSpec((nrel, nbasis), lambda i: (0, 0)),
            pl.BlockSpec((1, f_out), lambda i: (0, 0)),
        ],
        out_specs=[
            pl.BlockSpec((block_rows, f_out), lambda i: (i, 0)),
            pl.BlockSpec((nbasis, block_rows, n), lambda i: (0, i, 0)),
        ],
        out_shape=[
            jax.ShapeDtypeStruct((n, f_out), jnp.float32),
            jax.ShapeDtypeStruct((nbasis, n, n), jnp.bfloat16),
        ],
        compiler_params=pltpu.CompilerParams(
            dimension_semantics=("arbitrary",),
            vmem_limit_bytes=110 * 1024 * 1024,
        ),
    )(A, xw1, w_rel2, bias2d)


def _layer2_body(ab2_ref, hb_ref, b_ref, o_ref, *, nbasis):
    acc = jnp.dot(ab2_ref[0], hb_ref[0], preferred_element_type=jnp.float32)
    for b in range(1, nbasis):
        acc += jnp.dot(ab2_ref[b], hb_ref[b], preferred_element_type=jnp.float32)
    o_ref[...] = acc + b_ref[...]


def _layer2(Ab2, hb, bias2, block_rows=512):
    nbasis, n, _ = Ab2.shape
    f_out = hb.shape[2]
    bias2d = bias2.reshape(1, f_out)
    grid = (n // block_rows,)
    return pl.pallas_call(
        functools.partial(_layer2_body, nbasis=nbasis),
        grid=grid,
        in_specs=[
            pl.BlockSpec((nbasis, block_rows, n), lambda i: (0, i, 0)),
            pl.BlockSpec((nbasis, n, f_out), lambda i: (0, 0, 0)),
            pl.BlockSpec((1, f_out), lambda i: (0, 0)),
        ],
        out_specs=pl.BlockSpec((block_rows, f_out), lambda i: (i, 0)),
        out_shape=jax.ShapeDtypeStruct((n, f_out), jnp.float32),
        compiler_params=pltpu.CompilerParams(
            dimension_semantics=("arbitrary",),
        ),
    )(Ab2, hb, bias2d)


def kernel(A, x, w_bases1, w_rel1, bias1, w_bases2, w_rel2, bias2):
    xw1 = _project(x, w_bases1, w_rel1)
    h, Ab2 = _layer1(A, xw1, w_rel2, bias1)
    hb = _project_basis(h, w_bases2)
    return _layer2(Ab2, hb, bias2)


# speedup vs baseline: 1.4440x; 1.1671x over previous
import functools

import jax
import jax.numpy as jnp
from jax.experimental import pallas as pl
from jax.experimental.pallas import tpu as pltpu


def _proj_body(x_ref, wb_ref, wr_ref, xw_ref):
    x = x_ref[...]
    wr = wr_ref[...]
    nb = wb_ref.shape[0]
    nr = wr.shape[0]
    xb = [
        jnp.dot(x, wb_ref[b], preferred_element_type=jnp.float32)
        for b in range(nb)
    ]
    for r in range(nr):
        acc = wr[r, 0] * xb[0]
        for b in range(1, nb):
            acc = acc + wr[r, b] * xb[b]
        xw_ref[r] = acc.astype(jnp.bfloat16)


def _project(x, w_bases, w_rel):
    nbasis, _, f_out = w_bases.shape
    nrel = w_rel.shape[0]
    n = x.shape[0]
    return pl.pallas_call(
        _proj_body,
        out_shape=jax.ShapeDtypeStruct((nrel, n, f_out), jnp.bfloat16),
    )(x, w_bases, w_rel)


def _proj_basis_body(x_ref, wb_ref, hb_ref):
    nb = wb_ref.shape[0]
    x = x_ref[...]
    for b in range(nb):
        hb_ref[b] = jnp.dot(
            x, wb_ref[b], preferred_element_type=jnp.float32
        ).astype(jnp.bfloat16)


def _project_basis(x, w_bases):
    nbasis, _, f_out = w_bases.shape
    n = x.shape[0]
    return pl.pallas_call(
        _proj_basis_body,
        out_shape=jax.ShapeDtypeStruct((nbasis, n, f_out), jnp.bfloat16),
    )(x, w_bases)


def _layer1_body(a_ref, xw_ref, wr2_ref, b_ref, h_ref, ab2_ref, *, nrel, nbasis):
    ac = [a_ref[r].astype(jnp.bfloat16) for r in range(nrel)]
    acc = jnp.dot(ac[0], xw_ref[0], preferred_element_type=jnp.float32)
    for r in range(1, nrel):
        acc += jnp.dot(ac[r], xw_ref[r], preferred_element_type=jnp.float32)
    h_ref[...] = jnp.maximum(acc + b_ref[...], 0.0)
    wr2 = wr2_ref[...]
    for b in range(nbasis):
        combo = wr2[0, b].astype(jnp.bfloat16) * ac[0]
        for r in range(1, nrel):
            combo += wr2[r, b].astype(jnp.bfloat16) * ac[r]
        ab2_ref[b] = combo


def _layer1(A, xw1, w_rel2, bias1, block_rows=256):
    nrel, n, _ = A.shape
    f_out = xw1.shape[2]
    nbasis = w_rel2.shape[1]
    bias2d = bias1.reshape(1, f_out)
    grid = (n // block_rows,)
    return pl.pallas_call(
        functools.partial(_layer1_body, nrel=nrel, nbasis=nbasis),
        grid=grid,
        in_specs=[
            pl.BlockSpec((nrel, block_rows, n), lambda i: (0, i, 0)),
            pl.BlockSpec((nrel, n, f_out), lambda i: (0, 0, 0)),
            pl.BlockSpec((nrel, nbasis), lambda i: (0, 0)),
            pl.BlockSpec((1, f_out), lambda i: (0, 0)),
        ],
        out_specs=[
            pl.BlockSpec((block_rows, f_out), lambda i: (i, 0)),
            pl.BlockSpec((nbasis, block_rows, n), lambda i: (0, i, 0)),
        ],
        out_shape=[
            jax.ShapeDtypeStruct((n, f_out), jnp.float32),
            jax.ShapeDtypeStruct((nbasis, n, n), jnp.bfloat16),
        ],
        compiler_params=pltpu.CompilerParams(
            dimension_semantics=("arbitrary",),
            vmem_limit_bytes=110 * 1024 * 1024,
        ),
    )(A, xw1, w_rel2, bias2d)


def _layer2_body(ab2_ref, hb_ref, b_ref, o_ref, *, nbasis):
    acc = jnp.dot(ab2_ref[0], hb_ref[0], preferred_element_type=jnp.float32)
    for b in range(1, nbasis):
        acc += jnp.dot(ab2_ref[b], hb_ref[b], preferred_element_type=jnp.float32)
    o_ref[...] = acc + b_ref[...]


def _layer2(Ab2, hb, bias2, block_rows=512):
    nbasis, n, _ = Ab2.shape
    f_out = hb.shape[2]
    bias2d = bias2.reshape(1, f_out)
    grid = (n // block_rows,)
    return pl.pallas_call(
        functools.partial(_layer2_body, nbasis=nbasis),
        grid=grid,
        in_specs=[
            pl.BlockSpec((nbasis, block_rows, n), lambda i: (0, i, 0)),
            pl.BlockSpec((nbasis, n, f_out), lambda i: (0, 0, 0)),
            pl.BlockSpec((1, f_out), lambda i: (0, 0)),
        ],
        out_specs=pl.BlockSpec((block_rows, f_out), lambda i: (i, 0)),
        out_shape=jax.ShapeDtypeStruct((n, f_out), jnp.float32),
        compiler_params=pltpu.CompilerParams(
            dimension_semantics=("arbitrary",),
        ),
    )(Ab2, hb, bias2d)


def kernel(A, x, w_bases1, w_rel1, bias1, w_bases2, w_rel2, bias2):
    xw1 = _project(x, w_bases1, w_rel1)
    h, Ab2 = _layer1(A, xw1, w_rel2, bias1)
    hb = _project_basis(h, w_bases2)
    return _layer2(Ab2, hb, bias2)
